# baseline (speedup 1.0000x reference)
import functools

import jax
import jax.numpy as jnp
from jax import lax
from jax.experimental import pallas as pl
from jax.experimental.pallas import tpu as pltpu
from jax.experimental.pallas import tpu_sc as plsc

_NC = 2
_NS = 16
_NW = _NC * _NS
_G = 64


def _mm_body(x_ref, w_ref, b_ref, o_ref):
    o_ref[...] = (
        jnp.dot(x_ref[...], w_ref[...], preferred_element_type=jnp.float32, precision=lax.Precision.HIGHEST)
        + b_ref[...]
    )


def _mm(x, w, b, block_m):
    m, k = x.shape
    f = w.shape[1]
    assert m % block_m == 0
    return pl.pallas_call(
        _mm_body,
        grid=(m // block_m,),
        in_specs=[
            pl.BlockSpec((block_m, k), lambda i: (i, 0)),
            pl.BlockSpec((k, f), lambda i: (0, 0)),
            pl.BlockSpec((1, f), lambda i: (0, 0)),
        ],
        out_specs=pl.BlockSpec((block_m, f), lambda i: (i, 0)),
        out_shape=jax.ShapeDtypeStruct((m, f), jnp.float32),
    )(x, w, b.reshape(1, f))


def _update_body(acc_ref, degt_ref, h_ref, w2_ref, b2_ref, uw1h_ref, uw1a_ref,
                 ub1_ref, uw2_ref, ub2_ref, o_ref):
    acc = acc_ref[0] + acc_ref[1]
    deg = jnp.sum(degt_ref[...], axis=1, keepdims=True)
    agg = (jnp.dot(acc, w2_ref[...], preferred_element_type=jnp.float32, precision=lax.Precision.HIGHEST)
           + deg * b2_ref[...])
    u = jnp.maximum(
        jnp.dot(h_ref[...], uw1h_ref[...], preferred_element_type=jnp.float32, precision=lax.Precision.HIGHEST)
        + jnp.dot(agg, uw1a_ref[...], preferred_element_type=jnp.float32, precision=lax.Precision.HIGHEST)
        + ub1_ref[...], 0.0)
    h2 = jnp.dot(u, uw2_ref[...], preferred_element_type=jnp.float32, precision=lax.Precision.HIGHEST) + ub2_ref[...]
    o_ref[...] = jnp.maximum(h2, 0.0)


def _update(acc, degt, h, w2, b2, uw1h, uw1a, ub1, uw2, ub2, block_m):
    n, d = h.shape
    f = uw2.shape[1]
    return pl.pallas_call(
        _update_body,
        grid=(n // block_m,),
        in_specs=[
            pl.BlockSpec((2, block_m, d), lambda i: (0, i, 0)),
            pl.BlockSpec((block_m, _NC), lambda i: (i, 0)),
            pl.BlockSpec((block_m, d), lambda i: (i, 0)),
            pl.BlockSpec((d, f), lambda i: (0, 0)),
            pl.BlockSpec((1, f), lambda i: (0, 0)),
            pl.BlockSpec((d, f), lambda i: (0, 0)),
            pl.BlockSpec((d, f), lambda i: (0, 0)),
            pl.BlockSpec((1, f), lambda i: (0, 0)),
            pl.BlockSpec((d, f), lambda i: (0, 0)),
            pl.BlockSpec((1, f), lambda i: (0, 0)),
        ],
        out_specs=pl.BlockSpec((block_m, f), lambda i: (i, 0)),
        out_shape=jax.ShapeDtypeStruct((n, f), jnp.float32),
    )(acc, degt, h, w2, b2.reshape(1, f), uw1h, uw1a, ub1.reshape(1, f),
      uw2, ub2.reshape(1, f))


def _head_body(h_ref, bidx_ref, lw1_ref, lb1_ref, lw2_ref, lb2_ref, o_ref):
    bidx = bidx_ref[...]
    gids = lax.broadcasted_iota(jnp.int32, (_G, 1), 0)
    onehot = (bidx == gids).astype(jnp.float32)
    sums = jnp.dot(onehot, h_ref[...], preferred_element_type=jnp.float32, precision=lax.Precision.HIGHEST)
    counts = jnp.sum(onehot, axis=1, keepdims=True)
    pooled = sums / jnp.maximum(counts, 1.0)
    h2 = jnp.maximum(
        jnp.dot(pooled, lw1_ref[...], preferred_element_type=jnp.float32, precision=lax.Precision.HIGHEST)
        + lb1_ref[...], 0.0)
    o_ref[...] = jnp.sum(h2 * lw2_ref[...], axis=1, keepdims=True) + lb2_ref[...]


def _head(h, batch_idx, lw1, lb1, lw2, lb2):
    n, d = h.shape
    return pl.pallas_call(
        _head_body,
        out_shape=jax.ShapeDtypeStruct((_G, 1), jnp.float32),
    )(h, batch_idx.reshape(1, n), lw1, lb1.reshape(1, d),
      lw2.reshape(1, d), lb2.reshape(1, 1))


def _make_edge_kernel(n, e, d, chunk):
    e_per_w = e // _NW
    steps = e_per_w // chunk
    assert e % _NW == 0 and e_per_w % chunk == 0 and chunk % 16 == 0
    assert chunk <= 128
    rows_per_tile = (n // _NS) // 8 * 8
    tail_base = rows_per_tile * _NS
    tail = n - tail_base
    zr = 16
    assert rows_per_tile % zr == 0 and tail % 8 == 0 and tail <= zr
    mesh = plsc.VectorSubcoreMesh(core_axis_name="c", subcore_axis_name="s",
                                  num_cores=_NC, num_subcores=_NS)

    @functools.partial(
        pl.kernel,
        out_type=[
            jax.ShapeDtypeStruct((_NC, n, d), jnp.float32),
            jax.ShapeDtypeStruct((_NC * n,), jnp.float32),
        ],
        mesh=mesh,
        scratch_types=[
            pltpu.VMEM((chunk,), jnp.int32),
            pltpu.VMEM((chunk,), jnp.int32),
            pltpu.VMEM((chunk, d), jnp.float32),
            pltpu.VMEM((chunk, d), jnp.float32),
            pltpu.VMEM((chunk, d), jnp.float32),
            pltpu.VMEM((zr, d), jnp.float32),
            pltpu.VMEM((rows_per_tile + tail,), jnp.float32),
            pltpu.VMEM((chunk,), jnp.float32),
            pltpu.VMEM_SHARED((n, d), jnp.float32),
            pltpu.VMEM_SHARED((n,), jnp.float32),
            pltpu.SemaphoreType.DMA,
            pltpu.SemaphoreType.DMA,
        ],
    )
    def edge_kernel(pd_hbm, ps_hbm, pe_hbm, dst_hbm, src_hbm,
                    acc_hbm, deg_hbm,
                    idx_d, idx_s, pd_v, ps_v, pe_v, zbuf, zbuf1, ones_v,
                    acc_sh, deg_sh, sem_d, sem_s):
        cid = lax.axis_index("c")
        sid = lax.axis_index("s")
        wid = sid * _NC + cid

        zero16 = jnp.zeros((16,), jnp.float32)
        ones16 = jnp.ones((16,), jnp.float32)

        def zrow(i, _):
            def zcol(k, _):
                zbuf[i, pl.ds(k * 16, 16)] = zero16
                return 0
            return lax.fori_loop(0, d // 16, zcol, 0)
        lax.fori_loop(0, zr, zrow, 0)

        def zones(i, _):
            ones_v[pl.ds(i * 16, 16)] = ones16
            return 0
        lax.fori_loop(0, chunk // 16, zones, 0)

        def zvec(i, _):
            zbuf1[pl.ds(i * 16, 16)] = zero16
            return 0
        lax.fori_loop(0, (rows_per_tile + tail) // 16, zvec, 0)

        base_row = sid * rows_per_tile
        for q in range(rows_per_tile // zr):
            pltpu.sync_copy(zbuf, acc_sh.at[pl.ds(base_row + q * zr, zr)])

        @pl.when(sid < _NS - 1)
        def _zero_deg():
            pltpu.sync_copy(zbuf1.at[pl.ds(0, rows_per_tile)],
                            deg_sh.at[pl.ds(base_row, rows_per_tile)])

        @pl.when(sid == _NS - 1)
        def _zero_tail():
            pltpu.sync_copy(zbuf.at[pl.ds(0, tail)],
                            acc_sh.at[pl.ds(tail_base, tail)])
            pltpu.sync_copy(zbuf1,
                            deg_sh.at[pl.ds(base_row, rows_per_tile + tail)])
        plsc.subcore_barrier()

        def step(j, _):
            base = wid * e_per_w + j * chunk
            pltpu.sync_copy(dst_hbm.at[pl.ds(base, chunk)], idx_d)
            pltpu.sync_copy(src_hbm.at[pl.ds(base, chunk)], idx_s)
            cp_d = pltpu.async_copy(pd_hbm.at[idx_d], pd_v, sem_d)
            cp_s = pltpu.async_copy(ps_hbm.at[idx_s], ps_v, sem_s)
            pltpu.sync_copy(pe_hbm.at[pl.ds(base, chunk)], pe_v)
            pltpu.sync_copy(ones_v, deg_sh.at[idx_d], add=True)

            cp_d.wait()
            cp_s.wait()

            groups = d // 16

            @plsc.parallel_loop(0, chunk * groups, 1, unroll=8)
            def _ew(t):
                i = lax.div(t, groups)
                k = lax.rem(t, groups)
                s = pl.ds(k * 16, 16)
                v = pd_v[i, s] + ps_v[i, s] + pe_v[i, s]
                pd_v[i, s] = jnp.maximum(v, 0.0)

            pltpu.sync_copy(pd_v, acc_sh.at[idx_d], add=True)
            return 0
        lax.fori_loop(0, steps, step, 0)

        plsc.subcore_barrier()
        pltpu.sync_copy(acc_sh.at[pl.ds(base_row, rows_per_tile)],
                        acc_hbm.at[cid, pl.ds(base_row, rows_per_tile)])

        @pl.when(sid == _NS - 1)
        def _write_tail():
            pltpu.sync_copy(acc_sh.at[pl.ds(tail_base, tail)],
                            acc_hbm.at[cid, pl.ds(tail_base, tail)])
            pltpu.sync_copy(deg_sh.at[pl.ds(base_row, rows_per_tile + tail)],
                            zbuf1)
            pltpu.sync_copy(zbuf1,
                            deg_hbm.at[pl.ds(cid * n + base_row,
                                             rows_per_tile + tail)])

        @pl.when(sid < _NS - 1)
        def _write_deg():
            pltpu.sync_copy(deg_sh.at[pl.ds(base_row, rows_per_tile)],
                            zbuf1.at[pl.ds(0, rows_per_tile)])
            pltpu.sync_copy(
                zbuf1.at[pl.ds(0, rows_per_tile)],
                deg_hbm.at[pl.ds(cid * n + base_row, rows_per_tile)])

    return edge_kernel


def kernel(x, edge_index, edge_attr, batch_idx,
           m0_w1, m0_b1, m0_w2, m0_b2, u0_w1, u0_b1, u0_w2, u0_b2,
           m1_w1, m1_b1, m1_w2, m1_b2, u1_w1, u1_b1, u1_w2, u1_b2,
           lin1_w, lin1_b, lin2_w, lin2_b):
    n, d = x.shape
    e = edge_index.shape[1]
    h_dim = m0_w2.shape[0]
    src = edge_index[0]
    dst = edge_index[1]

    edge_fn = _make_edge_kernel(n, e, h_dim, chunk=16)

    def conv(h, mw1, mb1, mw2, mb2, uw1, ub1, uw2, ub2, degt):
        hd = h.shape[1]
        pd = _mm(h, mw1[:hd], mb1 * 0.0, block_m=2000)
        ps = _mm(h, mw1[hd:2 * hd], mb1 * 0.0, block_m=2000)
        pe = _mm(edge_attr, mw1[2 * hd:], mb1, block_m=8000)
        acc, deg = edge_fn(pd, ps, pe, dst, src)
        if degt is None:
            degt = jnp.transpose(deg.reshape(_NC, h.shape[0]))
        h_new = _update(acc, degt, h, mw2, mb2, uw1[:hd], uw1[hd:],
                        ub1, uw2, ub2, block_m=2000)
        return h_new, degt

    h1, degt = conv(x, m0_w1, m0_b1, m0_w2, m0_b2,
                    u0_w1, u0_b1, u0_w2, u0_b2, None)
    h2, _ = conv(h1, m1_w1, m1_b1, m1_w2, m1_b2,
                 u1_w1, u1_b1, u1_w2, u1_b2, degt)
    return _head(h2, batch_idx, lin1_w, lin1_b, lin2_w, lin2_b)


# speedup vs baseline: 1.7279x; 1.7279x over previous
import functools

import jax
import jax.numpy as jnp
from jax import lax
from jax.experimental import pallas as pl
from jax.experimental.pallas import tpu as pltpu
from jax.experimental.pallas import tpu_sc as plsc

_NC = 2
_NS = 16
_G = 64
_PREC = lax.Precision.HIGHEST


def _dot(a, b):
    return jnp.dot(a, b, preferred_element_type=jnp.float32, precision=_PREC)


def _mm_split_body(x_ref, w_ref, o_ref):
    o_ref[0] = _dot(x_ref[...], w_ref[0])


def _mm_split(x, w):
    m, k = x.shape
    f = w.shape[1]
    f2 = f // _NC
    ws = jnp.stack([w[:, i * f2:(i + 1) * f2] for i in range(_NC)])
    return pl.pallas_call(
        _mm_split_body,
        grid=(_NC,),
        in_specs=[
            pl.BlockSpec((m, k), lambda c: (0, 0)),
            pl.BlockSpec((1, k, f2), lambda c: (c, 0, 0)),
        ],
        out_specs=pl.BlockSpec((1, m, f2), lambda c: (c, 0, 0)),
        out_shape=jax.ShapeDtypeStruct((_NC, m, f2), jnp.float32),
    )(x, ws)


def _mm_pe_body(x_ref, w_ref, b_ref, o_ref):
    o_ref[0] = _dot(x_ref[...], w_ref[0]) + b_ref[0]


def _mm_pe(x, w, b, block_m):
    m, k = x.shape
    f = w.shape[1]
    f2 = f // _NC
    assert m % block_m == 0
    ws = jnp.stack([w[:, i * f2:(i + 1) * f2] for i in range(_NC)])
    return pl.pallas_call(
        _mm_pe_body,
        grid=(_NC, m // block_m),
        in_specs=[
            pl.BlockSpec((block_m, k), lambda c, i: (i, 0)),
            pl.BlockSpec((1, k, f2), lambda c, i: (c, 0, 0)),
            pl.BlockSpec((1, 1, f2), lambda c, i: (c, 0, 0)),
        ],
        out_specs=pl.BlockSpec((1, block_m, f2), lambda c, i: (c, i, 0)),
        out_shape=jax.ShapeDtypeStruct((_NC, m, f2), jnp.float32),
    )(x, ws, b.reshape(_NC, 1, f2))


def _update_body(acc_ref, deg_ref, h_ref, w2_ref, b2_ref, uw1h_ref, uw1a_ref,
                 ub1_ref, uw2_ref, ub2_ref, o_ref):
    dh = acc_ref.shape[2]
    agg = (_dot(acc_ref[0], w2_ref[0:dh, :])
           + _dot(acc_ref[1], w2_ref[dh:2 * dh, :])
           + deg_ref[...] * b2_ref[...])
    u = jnp.maximum(
        _dot(h_ref[...], uw1h_ref[...]) + _dot(agg, uw1a_ref[...])
        + ub1_ref[...], 0.0)
    h2 = _dot(u, uw2_ref[...]) + ub2_ref[...]
    o_ref[...] = jnp.maximum(h2, 0.0)


def _update(acc, deg, h, w2, b2, uw1h, uw1a, ub1, uw2, ub2, block_m):
    n, d = h.shape
    d2 = d // _NC
    f = uw2.shape[1]
    return pl.pallas_call(
        _update_body,
        grid=(n // block_m,),
        in_specs=[
            pl.BlockSpec((_NC, block_m, d2), lambda i: (0, i, 0)),
            pl.BlockSpec((block_m, 1), lambda i: (i, 0)),
            pl.BlockSpec((block_m, d), lambda i: (i, 0)),
            pl.BlockSpec((d, f), lambda i: (0, 0)),
            pl.BlockSpec((1, f), lambda i: (0, 0)),
            pl.BlockSpec((d, f), lambda i: (0, 0)),
            pl.BlockSpec((d, f), lambda i: (0, 0)),
            pl.BlockSpec((1, f), lambda i: (0, 0)),
            pl.BlockSpec((d, f), lambda i: (0, 0)),
            pl.BlockSpec((1, f), lambda i: (0, 0)),
        ],
        out_specs=pl.BlockSpec((block_m, f), lambda i: (i, 0)),
        out_shape=jax.ShapeDtypeStruct((n, f), jnp.float32),
    )(acc, deg, h, w2, b2.reshape(1, f), uw1h, uw1a, ub1.reshape(1, f),
      uw2, ub2.reshape(1, f))


def _head_body(h_ref, bidx_ref, lw1_ref, lb1_ref, lw2_ref, lb2_ref, o_ref):
    bidx = bidx_ref[...]
    gids = lax.broadcasted_iota(jnp.int32, (_G, 1), 0)
    onehot = (bidx == gids).astype(jnp.float32)
    sums = _dot(onehot, h_ref[...])
    counts = jnp.sum(onehot, axis=1, keepdims=True)
    pooled = sums / jnp.maximum(counts, 1.0)
    h2 = jnp.maximum(_dot(pooled, lw1_ref[...]) + lb1_ref[...], 0.0)
    o_ref[...] = jnp.sum(h2 * lw2_ref[...], axis=1, keepdims=True) + lb2_ref[...]


def _head(h, batch_idx, lw1, lb1, lw2, lb2):
    n, d = h.shape
    return pl.pallas_call(
        _head_body,
        out_shape=jax.ShapeDtypeStruct((_G, 1), jnp.float32),
    )(h, batch_idx.reshape(1, n), lw1, lb1.reshape(1, d),
      lw2.reshape(1, d), lb2.reshape(1, 1))


_NBUF = 2
_NIDX = 4


def _make_edge_kernel(n, e, d, chunk):
    dh = d // _NC
    e_per_t = e // _NS
    steps = e_per_t // chunk
    assert e % _NS == 0 and e_per_t % chunk == 0 and chunk % 16 == 0
    assert chunk <= 128
    assert steps % 2 == 0 and steps >= 2 * _NIDX
    rows_per_tile = (n // _NS) // 8 * 8
    tail_base = rows_per_tile * _NS
    tail = n - tail_base
    zr = 16
    assert rows_per_tile % zr == 0 and tail % 8 == 0 and tail <= zr
    groups = chunk * dh // 16
    mesh = plsc.VectorSubcoreMesh(core_axis_name="c", subcore_axis_name="s",
                                  num_cores=_NC, num_subcores=_NS)

    @functools.partial(
        pl.kernel,
        out_type=[
            jax.ShapeDtypeStruct((_NC, n, dh), jnp.float32),
            jax.ShapeDtypeStruct((n,), jnp.float32),
        ],
        mesh=mesh,
        compiler_params=pltpu.CompilerParams(use_tc_tiling_on_sc=False),
        scratch_types=(
            [pltpu.VMEM((chunk,), jnp.int32) for _ in range(_NIDX)]
            + [pltpu.VMEM((chunk,), jnp.int32) for _ in range(_NIDX)]
            + [pltpu.VMEM((chunk, dh), jnp.float32) for _ in range(3 * _NBUF)]
            + [
                pltpu.VMEM((zr, dh), jnp.float32),
                pltpu.VMEM((rows_per_tile + tail,), jnp.float32),
                pltpu.VMEM((chunk,), jnp.float32),
                pltpu.VMEM_SHARED((n, dh), jnp.float32),
                pltpu.VMEM_SHARED((n,), jnp.float32),
            ]
            + [pltpu.SemaphoreType.DMA for _ in range(3 * _NBUF + 2 * _NIDX)]
        ),
    )
    def edge_kernel(pd_hbm, ps_hbm, pe_hbm, dst_hbm, src_hbm,
                    acc_hbm, deg_hbm, *refs):
        idx_d = list(refs[0:_NIDX])
        idx_s = list(refs[_NIDX:2 * _NIDX])
        off = 2 * _NIDX
        pd_v = list(refs[off:off + _NBUF]); off += _NBUF
        ps_v = list(refs[off:off + _NBUF]); off += _NBUF
        pe_v = list(refs[off:off + _NBUF]); off += _NBUF
        zbuf, zbuf1, ones_v, acc_sh, deg_sh = refs[off:off + 5]; off += 5
        sem_pd = list(refs[off:off + _NBUF]); off += _NBUF
        sem_ps = list(refs[off:off + _NBUF]); off += _NBUF
        sem_pe = list(refs[off:off + _NBUF]); off += _NBUF
        sem_id = list(refs[off:off + _NIDX]); off += _NIDX
        sem_is = list(refs[off:off + _NIDX]); off += _NIDX

        cid = lax.axis_index("c")
        sid = lax.axis_index("s")
        pd_t = pd_hbm.at[cid]
        ps_t = ps_hbm.at[cid]
        pe_t = pe_hbm.at[cid]
        tile_base = sid * e_per_t

        zero16 = jnp.zeros((16,), jnp.float32)
        ones16 = jnp.ones((16,), jnp.float32)

        def zrow(i, _):
            def zcol(k, _):
                zbuf[i, pl.ds(k * 16, 16)] = zero16
                return 0
            return lax.fori_loop(0, dh // 16, zcol, 0)
        lax.fori_loop(0, zr, zrow, 0)

        def zones(i, _):
            ones_v[pl.ds(i * 16, 16)] = ones16
            return 0
        lax.fori_loop(0, chunk // 16, zones, 0)

        def zvec(i, _):
            zbuf1[pl.ds(i * 16, 16)] = zero16
            return 0
        lax.fori_loop(0, (rows_per_tile + tail) // 16, zvec, 0)

        base_row = sid * rows_per_tile
        for q in range(rows_per_tile // zr):
            pltpu.sync_copy(zbuf, acc_sh.at[pl.ds(base_row + q * zr, zr)])

        @pl.when((sid == _NS - 1) & (cid == 0))
        def _zero_deg_tail():
            pltpu.sync_copy(zbuf.at[pl.ds(0, tail)],
                            acc_sh.at[pl.ds(tail_base, tail)])
            pltpu.sync_copy(zbuf1,
                            deg_sh.at[pl.ds(base_row, rows_per_tile + tail)])

        @pl.when((sid == _NS - 1) & (cid == 1))
        def _zero_acc_tail():
            pltpu.sync_copy(zbuf.at[pl.ds(0, tail)],
                            acc_sh.at[pl.ds(tail_base, tail)])

        @pl.when((sid < _NS - 1) & (cid == 0))
        def _zero_deg():
            pltpu.sync_copy(zbuf1.at[pl.ds(0, rows_per_tile)],
                            deg_sh.at[pl.ds(base_row, rows_per_tile)])
        plsc.subcore_barrier()

        def idx_load(j, q):
            base = tile_base + j * chunk
            pltpu.async_copy(dst_hbm.at[pl.ds(base, chunk)], idx_d[q],
                             sem_id[q])
            pltpu.async_copy(src_hbm.at[pl.ds(base, chunk)], idx_s[q],
                             sem_is[q])

        def prep(j, b, q, load_ahead):
            base = tile_base + j * chunk
            pltpu.make_async_copy(dst_hbm.at[pl.ds(base, chunk)], idx_d[q],
                                  sem_id[q]).wait()
            pltpu.make_async_copy(src_hbm.at[pl.ds(base, chunk)], idx_s[q],
                                  sem_is[q]).wait()

            @pl.when(cid == 0)
            def _deg():
                pltpu.sync_copy(ones_v, deg_sh.at[idx_d[q]], add=True)
            pltpu.async_copy(pd_t.at[idx_d[q]], pd_v[b], sem_pd[b])
            pltpu.async_copy(ps_t.at[idx_s[q]], ps_v[b], sem_ps[b])
            pltpu.async_copy(pe_t.at[pl.ds(base, chunk)], pe_v[b], sem_pe[b])
            if load_ahead:
                idx_load(j + 2, (q + 2) % _NIDX)

        def finish(j, b, q):
            base = tile_base + j * chunk
            pltpu.make_async_copy(pd_t.at[idx_d[q]], pd_v[b],
                                  sem_pd[b]).wait()
            pltpu.make_async_copy(ps_t.at[idx_s[q]], ps_v[b],
                                  sem_ps[b]).wait()
            pltpu.make_async_copy(pe_t.at[pl.ds(base, chunk)], pe_v[b],
                                  sem_pe[b]).wait()

            gpr = dh // 16

            @plsc.parallel_loop(0, groups, 1, unroll=8)
            def _ew(t):
                i = lax.div(t, gpr)
                k = lax.rem(t, gpr)
                s = pl.ds(k * 16, 16)
                v = pd_v[b][i, s] + ps_v[b][i, s] + pe_v[b][i, s]
                pd_v[b][i, s] = jnp.maximum(v, 0.0)

            pltpu.async_copy(pd_v[b], acc_sh.at[idx_d[q]], sem_pd[b],
                             add=True)
            pltpu.make_async_copy(pd_v[b], acc_sh.at[idx_d[q]],
                                  sem_pd[b]).wait()

        assert steps % 4 == 2 and steps >= 10
        for q in range(_NIDX):
            idx_load(q, q)
        prep(0, 0, 0, False)
        prep(1, 1, 1, False)
        finish(0, 0, 0)
        prep(2, 0, 2, True)
        finish(1, 1, 1)
        prep(3, 1, 3, True)

        def main_body(jj, _):
            j0 = 2 + jj * 4
            for p in range(4):
                j = j0 + p
                b = p % 2
                q = (2 + p) % _NIDX
                finish(j, b, q)
                prep(j + 2, b, (q + 2) % _NIDX, True)
            return 0
        lax.fori_loop(0, (steps - 6) // 4, main_body, 0)

        finish(steps - 4, 0, 2)
        prep(steps - 2, 0, 0, False)
        finish(steps - 3, 1, 3)
        prep(steps - 1, 1, 1, False)
        finish(steps - 2, 0, 0)
        finish(steps - 1, 1, 1)

        plsc.subcore_barrier()
        pltpu.sync_copy(acc_sh.at[pl.ds(base_row, rows_per_tile)],
                        acc_hbm.at[cid, pl.ds(base_row, rows_per_tile)])

        @pl.when(sid == _NS - 1)
        def _write_tail():
            pltpu.sync_copy(acc_sh.at[pl.ds(tail_base, tail)],
                            acc_hbm.at[cid, pl.ds(tail_base, tail)])

        @pl.when((sid == _NS - 1) & (cid == 0))
        def _write_deg_tail():
            pltpu.sync_copy(deg_sh.at[pl.ds(base_row, rows_per_tile + tail)],
                            zbuf1)
            pltpu.sync_copy(zbuf1,
                            deg_hbm.at[pl.ds(base_row, rows_per_tile + tail)])

        @pl.when((sid < _NS - 1) & (cid == 0))
        def _write_deg():
            pltpu.sync_copy(deg_sh.at[pl.ds(base_row, rows_per_tile)],
                            zbuf1.at[pl.ds(0, rows_per_tile)])
            pltpu.sync_copy(zbuf1.at[pl.ds(0, rows_per_tile)],
                            deg_hbm.at[pl.ds(base_row, rows_per_tile)])

    return edge_kernel


def kernel(x, edge_index, edge_attr, batch_idx,
           m0_w1, m0_b1, m0_w2, m0_b2, u0_w1, u0_b1, u0_w2, u0_b2,
           m1_w1, m1_b1, m1_w2, m1_b2, u1_w1, u1_b1, u1_w2, u1_b2,
           lin1_w, lin1_b, lin2_w, lin2_b):
    n, d = x.shape
    e = edge_index.shape[1]
    h_dim = m0_w2.shape[0]
    src = edge_index[0]
    dst = edge_index[1]

    edge_fn = _make_edge_kernel(n, e, h_dim, chunk=80)

    def conv(h, mw1, mb1, mw2, mb2, uw1, ub1, uw2, ub2, deg1):
        hd = h.shape[1]
        pd = _mm_split(h, mw1[:hd])
        ps = _mm_split(h, mw1[hd:2 * hd])
        pe = _mm_pe(edge_attr, mw1[2 * hd:], mb1, block_m=8000)
        acc, deg = edge_fn(pd, ps, pe, dst, src)
        if deg1 is None:
            deg1 = deg.reshape(n, 1)
        h_new = _update(acc, deg1, h, mw2, mb2, uw1[:hd], uw1[hd:],
                        ub1, uw2, ub2, block_m=2000)
        return h_new, deg1

    h1, deg1 = conv(x, m0_w1, m0_b1, m0_w2, m0_b2,
                    u0_w1, u0_b1, u0_w2, u0_b2, None)
    h2, _ = conv(h1, m1_w1, m1_b1, m1_w2, m1_b2,
                 u1_w1, u1_b1, u1_w2, u1_b2, deg1)
    return _head(h2, batch_idx, lin1_w, lin1_b, lin2_w, lin2_b)


# speedup vs baseline: 2.6466x; 1.5317x over previous
import functools

import jax
import jax.numpy as jnp
from jax import lax
from jax.experimental import pallas as pl
from jax.experimental.pallas import tpu as pltpu
from jax.experimental.pallas import tpu_sc as plsc

_NC = 2
_NS = 16
_G = 64
_PREC = lax.Precision.HIGHEST


def _dot(a, b):
    return jnp.dot(a, b, preferred_element_type=jnp.float32, precision=_PREC)


def _mm_split_body(x_ref, w_ref, o_ref):
    o_ref[0] = _dot(x_ref[...], w_ref[0])


def _mm_split(x, w):
    m, k = x.shape
    f = w.shape[1]
    f2 = f // _NC
    ws = jnp.stack([w[:, i * f2:(i + 1) * f2] for i in range(_NC)])
    return pl.pallas_call(
        _mm_split_body,
        grid=(_NC,),
        in_specs=[
            pl.BlockSpec((m, k), lambda c: (0, 0)),
            pl.BlockSpec((1, k, f2), lambda c: (c, 0, 0)),
        ],
        out_specs=pl.BlockSpec((1, m, f2), lambda c: (c, 0, 0)),
        out_shape=jax.ShapeDtypeStruct((_NC, m, f2), jnp.float32),
    )(x, ws)


def _mm_pe_body(x_ref, w_ref, b_ref, o_ref):
    o_ref[0] = _dot(x_ref[...], w_ref[0]) + b_ref[0]


def _mm_pe(x, w, b, block_m):
    m, k = x.shape
    f = w.shape[1]
    f2 = f // _NC
    assert m % (2 * block_m) == 0
    xp = x.reshape(m // 2, 2 * k)
    zero = jnp.zeros((k, f2), jnp.float32)
    ws = jnp.stack([
        jnp.concatenate([
            jnp.concatenate([w[:, i * f2:(i + 1) * f2], zero], axis=1),
            jnp.concatenate([zero, w[:, i * f2:(i + 1) * f2]], axis=1),
        ], axis=0)
        for i in range(_NC)
    ])
    bp = jnp.stack([jnp.tile(b[i * f2:(i + 1) * f2], 2) for i in range(_NC)])
    return pl.pallas_call(
        _mm_pe_body,
        grid=(_NC, m // 2 // block_m),
        in_specs=[
            pl.BlockSpec((block_m, 2 * k), lambda c, i: (i, 0)),
            pl.BlockSpec((1, 2 * k, 2 * f2), lambda c, i: (c, 0, 0)),
            pl.BlockSpec((1, 1, 2 * f2), lambda c, i: (c, 0, 0)),
        ],
        out_specs=pl.BlockSpec((1, block_m, 2 * f2), lambda c, i: (c, i, 0)),
        out_shape=jax.ShapeDtypeStruct((_NC, m // 2, 2 * f2), jnp.float32),
    )(xp, ws, bp.reshape(_NC, 1, 2 * f2))


def _update_body(acc_ref, deg_ref, h_ref, w2_ref, b2_ref, uw1h_ref, uw1a_ref,
                 ub1_ref, uw2_ref, ub2_ref, o_ref):
    dh = acc_ref.shape[2]
    agg = (_dot(acc_ref[0], w2_ref[0:dh, :])
           + _dot(acc_ref[1], w2_ref[dh:2 * dh, :])
           + deg_ref[...] * b2_ref[...])
    u = jnp.maximum(
        _dot(h_ref[...], uw1h_ref[...]) + _dot(agg, uw1a_ref[...])
        + ub1_ref[...], 0.0)
    h2 = _dot(u, uw2_ref[...]) + ub2_ref[...]
    o_ref[...] = jnp.maximum(h2, 0.0)


def _update(acc, deg, h, w2, b2, uw1h, uw1a, ub1, uw2, ub2, block_m):
    n, d = h.shape
    d2 = d // _NC
    f = uw2.shape[1]
    return pl.pallas_call(
        _update_body,
        grid=(n // block_m,),
        in_specs=[
            pl.BlockSpec((_NC, block_m, d2), lambda i: (0, i, 0)),
            pl.BlockSpec((block_m, 1), lambda i: (i, 0)),
            pl.BlockSpec((block_m, d), lambda i: (i, 0)),
            pl.BlockSpec((d, f), lambda i: (0, 0)),
            pl.BlockSpec((1, f), lambda i: (0, 0)),
            pl.BlockSpec((d, f), lambda i: (0, 0)),
            pl.BlockSpec((d, f), lambda i: (0, 0)),
            pl.BlockSpec((1, f), lambda i: (0, 0)),
            pl.BlockSpec((d, f), lambda i: (0, 0)),
            pl.BlockSpec((1, f), lambda i: (0, 0)),
        ],
        out_specs=pl.BlockSpec((block_m, f), lambda i: (i, 0)),
        out_shape=jax.ShapeDtypeStruct((n, f), jnp.float32),
    )(acc, deg, h, w2, b2.reshape(1, f), uw1h, uw1a, ub1.reshape(1, f),
      uw2, ub2.reshape(1, f))


def _head_body(h_ref, bidx_ref, lw1_ref, lb1_ref, lw2_ref, lb2_ref, o_ref):
    bidx = bidx_ref[...]
    gids = lax.broadcasted_iota(jnp.int32, (_G, 1), 0)
    onehot = (bidx == gids).astype(jnp.float32)
    sums = _dot(onehot, h_ref[...])
    counts = jnp.sum(onehot, axis=1, keepdims=True)
    pooled = sums / jnp.maximum(counts, 1.0)
    h2 = jnp.maximum(_dot(pooled, lw1_ref[...]) + lb1_ref[...], 0.0)
    o_ref[...] = jnp.sum(h2 * lw2_ref[...], axis=1, keepdims=True) + lb2_ref[...]


def _head(h, batch_idx, lw1, lb1, lw2, lb2):
    n, d = h.shape
    return pl.pallas_call(
        _head_body,
        out_shape=jax.ShapeDtypeStruct((_G, 1), jnp.float32),
    )(h, batch_idx.reshape(1, n), lw1, lb1.reshape(1, d),
      lw2.reshape(1, d), lb2.reshape(1, 1))


_NBUF = 2
_NIDX = 4


def _make_edge_kernel(n, e, d, chunk):
    dh = d // _NC
    e_per_t = e // _NS
    steps = e_per_t // chunk
    assert e % _NS == 0 and e_per_t % chunk == 0 and chunk % 16 == 0
    assert chunk <= 128
    assert steps % 2 == 0 and steps >= 2 * _NIDX
    rows_per_tile = (n // _NS) // 8 * 8
    tail_base = rows_per_tile * _NS
    tail = n - tail_base
    zr = 16
    assert rows_per_tile % zr == 0 and tail % 8 == 0 and tail <= zr
    groups = chunk * dh // 16
    mesh = plsc.VectorSubcoreMesh(core_axis_name="c", subcore_axis_name="s",
                                  num_cores=_NC, num_subcores=_NS)

    @functools.partial(
        pl.kernel,
        out_type=[
            jax.ShapeDtypeStruct((_NC, n, dh), jnp.float32),
            jax.ShapeDtypeStruct((n,), jnp.float32),
        ],
        mesh=mesh,
        compiler_params=pltpu.CompilerParams(use_tc_tiling_on_sc=False),
        scratch_types=(
            [pltpu.VMEM((chunk,), jnp.int32) for _ in range(_NIDX)]
            + [pltpu.VMEM((chunk,), jnp.int32) for _ in range(_NIDX)]
            + [pltpu.VMEM((chunk, dh), jnp.float32) for _ in range(2 * _NBUF)]
            + [pltpu.VMEM((chunk // 2, 2 * dh), jnp.float32) for _ in range(_NBUF)]
            + [
                pltpu.VMEM((zr, dh), jnp.float32),
                pltpu.VMEM((rows_per_tile + tail,), jnp.float32),
                pltpu.VMEM((chunk,), jnp.float32),
                pltpu.VMEM_SHARED((n, dh), jnp.float32),
                pltpu.VMEM_SHARED((n,), jnp.float32),
            ]
            + [pltpu.SemaphoreType.DMA for _ in range(3 * _NBUF + 2 * _NIDX)]
        ),
    )
    def edge_kernel(pd_hbm, ps_hbm, pe_hbm, dst_hbm, src_hbm,
                    acc_hbm, deg_hbm, *refs):
        idx_d = list(refs[0:_NIDX])
        idx_s = list(refs[_NIDX:2 * _NIDX])
        off = 2 * _NIDX
        pd_v = list(refs[off:off + _NBUF]); off += _NBUF
        ps_v = list(refs[off:off + _NBUF]); off += _NBUF
        pe_v = list(refs[off:off + _NBUF]); off += _NBUF
        zbuf, zbuf1, ones_v, acc_sh, deg_sh = refs[off:off + 5]; off += 5
        sem_pd = list(refs[off:off + _NBUF]); off += _NBUF
        sem_ps = list(refs[off:off + _NBUF]); off += _NBUF
        sem_pe = list(refs[off:off + _NBUF]); off += _NBUF
        sem_id = list(refs[off:off + _NIDX]); off += _NIDX
        sem_is = list(refs[off:off + _NIDX]); off += _NIDX

        cid = lax.axis_index("c")
        sid = lax.axis_index("s")
        pd_t = pd_hbm.at[cid]
        ps_t = ps_hbm.at[cid]
        pe_t = pe_hbm.at[cid]
        tile_base = sid * e_per_t

        zero16 = jnp.zeros((16,), jnp.float32)
        ones16 = jnp.ones((16,), jnp.float32)

        def zrow(i, _):
            def zcol(k, _):
                zbuf[i, pl.ds(k * 16, 16)] = zero16
                return 0
            return lax.fori_loop(0, dh // 16, zcol, 0)
        lax.fori_loop(0, zr, zrow, 0)

        def zones(i, _):
            ones_v[pl.ds(i * 16, 16)] = ones16
            return 0
        lax.fori_loop(0, chunk // 16, zones, 0)

        def zvec(i, _):
            zbuf1[pl.ds(i * 16, 16)] = zero16
            return 0
        lax.fori_loop(0, (rows_per_tile + tail) // 16, zvec, 0)

        base_row = sid * rows_per_tile
        for q in range(rows_per_tile // zr):
            pltpu.sync_copy(zbuf, acc_sh.at[pl.ds(base_row + q * zr, zr)])

        @pl.when((sid == _NS - 1) & (cid == 0))
        def _zero_deg_tail():
            pltpu.sync_copy(zbuf.at[pl.ds(0, tail)],
                            acc_sh.at[pl.ds(tail_base, tail)])
            pltpu.sync_copy(zbuf1,
                            deg_sh.at[pl.ds(base_row, rows_per_tile + tail)])

        @pl.when((sid == _NS - 1) & (cid == 1))
        def _zero_acc_tail():
            pltpu.sync_copy(zbuf.at[pl.ds(0, tail)],
                            acc_sh.at[pl.ds(tail_base, tail)])

        @pl.when((sid < _NS - 1) & (cid == 0))
        def _zero_deg():
            pltpu.sync_copy(zbuf1.at[pl.ds(0, rows_per_tile)],
                            deg_sh.at[pl.ds(base_row, rows_per_tile)])
        plsc.subcore_barrier()

        def idx_load(j, q):
            base = tile_base + j * chunk
            pltpu.async_copy(dst_hbm.at[pl.ds(base, chunk)], idx_d[q],
                             sem_id[q])
            pltpu.async_copy(src_hbm.at[pl.ds(base, chunk)], idx_s[q],
                             sem_is[q])

        def prep(j, b, q, load_ahead):
            base = tile_base + j * chunk
            pltpu.make_async_copy(dst_hbm.at[pl.ds(base, chunk)], idx_d[q],
                                  sem_id[q]).wait()
            pltpu.make_async_copy(src_hbm.at[pl.ds(base, chunk)], idx_s[q],
                                  sem_is[q]).wait()

            @pl.when(cid == 0)
            def _deg():
                pltpu.sync_copy(ones_v, deg_sh.at[idx_d[q]], add=True)
            pltpu.async_copy(pd_t.at[idx_d[q]], pd_v[b], sem_pd[b])
            pltpu.async_copy(ps_t.at[idx_s[q]], ps_v[b], sem_ps[b])
            pltpu.async_copy(pe_t.at[pl.ds(base // 2, chunk // 2)], pe_v[b],
                             sem_pe[b])
            if load_ahead:
                idx_load(j + 2, (q + 2) % _NIDX)

        def finish(j, b, q):
            base = tile_base + j * chunk
            pltpu.make_async_copy(pd_t.at[idx_d[q]], pd_v[b],
                                  sem_pd[b]).wait()
            pltpu.make_async_copy(ps_t.at[idx_s[q]], ps_v[b],
                                  sem_ps[b]).wait()
            pltpu.make_async_copy(pe_t.at[pl.ds(base // 2, chunk // 2)],
                                  pe_v[b], sem_pe[b]).wait()

            gpr = dh // 16

            @plsc.parallel_loop(0, groups, 1, unroll=8)
            def _ew(t):
                i = lax.div(t, gpr)
                k = lax.rem(t, gpr)
                s = pl.ds(k * 16, 16)
                pe = pe_v[b][lax.div(t, 2 * gpr),
                             pl.ds(lax.rem(t, 2 * gpr) * 16, 16)]
                v = pd_v[b][i, s] + ps_v[b][i, s] + pe
                pd_v[b][i, s] = jnp.maximum(v, 0.0)

            pltpu.async_copy(pd_v[b], acc_sh.at[idx_d[q]], sem_pd[b],
                             add=True)
            pltpu.make_async_copy(pd_v[b], acc_sh.at[idx_d[q]],
                                  sem_pd[b]).wait()

        assert steps % 4 == 2 and steps >= 10
        for q in range(_NIDX):
            idx_load(q, q)
        prep(0, 0, 0, False)
        prep(1, 1, 1, False)
        finish(0, 0, 0)
        prep(2, 0, 2, True)
        finish(1, 1, 1)
        prep(3, 1, 3, True)

        def main_body(jj, _):
            j0 = 2 + jj * 4
            for p in range(4):
                j = j0 + p
                b = p % 2
                q = (2 + p) % _NIDX
                finish(j, b, q)
                prep(j + 2, b, (q + 2) % _NIDX, True)
            return 0
        lax.fori_loop(0, (steps - 6) // 4, main_body, 0)

        finish(steps - 4, 0, 2)
        prep(steps - 2, 0, 0, False)
        finish(steps - 3, 1, 3)
        prep(steps - 1, 1, 1, False)
        finish(steps - 2, 0, 0)
        finish(steps - 1, 1, 1)

        plsc.subcore_barrier()
        pltpu.sync_copy(acc_sh.at[pl.ds(base_row, rows_per_tile)],
                        acc_hbm.at[cid, pl.ds(base_row, rows_per_tile)])

        @pl.when(sid == _NS - 1)
        def _write_tail():
            pltpu.sync_copy(acc_sh.at[pl.ds(tail_base, tail)],
                            acc_hbm.at[cid, pl.ds(tail_base, tail)])

        @pl.when((sid == _NS - 1) & (cid == 0))
        def _write_deg_tail():
            pltpu.sync_copy(deg_sh.at[pl.ds(base_row, rows_per_tile + tail)],
                            zbuf1)
            pltpu.sync_copy(zbuf1,
                            deg_hbm.at[pl.ds(base_row, rows_per_tile + tail)])

        @pl.when((sid < _NS - 1) & (cid == 0))
        def _write_deg():
            pltpu.sync_copy(deg_sh.at[pl.ds(base_row, rows_per_tile)],
                            zbuf1.at[pl.ds(0, rows_per_tile)])
            pltpu.sync_copy(zbuf1.at[pl.ds(0, rows_per_tile)],
                            deg_hbm.at[pl.ds(base_row, rows_per_tile)])

    return edge_kernel


def kernel(x, edge_index, edge_attr, batch_idx,
           m0_w1, m0_b1, m0_w2, m0_b2, u0_w1, u0_b1, u0_w2, u0_b2,
           m1_w1, m1_b1, m1_w2, m1_b2, u1_w1, u1_b1, u1_w2, u1_b2,
           lin1_w, lin1_b, lin2_w, lin2_b):
    n, d = x.shape
    e = edge_index.shape[1]
    h_dim = m0_w2.shape[0]
    src = edge_index[0]
    dst = edge_index[1]

    edge_fn = _make_edge_kernel(n, e, h_dim, chunk=80)

    def conv(h, mw1, mb1, mw2, mb2, uw1, ub1, uw2, ub2, deg1):
        hd = h.shape[1]
        pd = _mm_split(h, mw1[:hd])
        ps = _mm_split(h, mw1[hd:2 * hd])
        pe = _mm_pe(edge_attr, mw1[2 * hd:], mb1, block_m=8000)
        acc, deg = edge_fn(pd, ps, pe, dst, src)
        if deg1 is None:
            deg1 = deg.reshape(n, 1)
        h_new = _update(acc, deg1, h, mw2, mb2, uw1[:hd], uw1[hd:],
                        ub1, uw2, ub2, block_m=2000)
        return h_new, deg1

    h1, deg1 = conv(x, m0_w1, m0_b1, m0_w2, m0_b2,
                    u0_w1, u0_b1, u0_w2, u0_b2, None)
    h2, _ = conv(h1, m1_w1, m1_b1, m1_w2, m1_b2,
                 u1_w1, u1_b1, u1_w2, u1_b2, deg1)
    return _head(h2, batch_idx, lin1_w, lin1_b, lin2_w, lin2_b)


# speedup vs baseline: 2.9636x; 1.1198x over previous
import functools

import jax
import jax.numpy as jnp
from jax import lax
from jax.experimental import pallas as pl
from jax.experimental.pallas import tpu as pltpu
from jax.experimental.pallas import tpu_sc as plsc

_NC = 2
_NS = 16
_G = 64
_PREC = lax.Precision.DEFAULT


def _dot(a, b):
    return jnp.dot(a, b, preferred_element_type=jnp.float32, precision=_PREC)


def _mm_split_body(x_ref, w_ref, o_ref):
    o_ref[0] = _dot(x_ref[...], w_ref[0])


def _mm_split(x, w):
    m, k = x.shape
    f = w.shape[1]
    f2 = f // _NC
    ws = jnp.stack([w[:, i * f2:(i + 1) * f2] for i in range(_NC)])
    return pl.pallas_call(
        _mm_split_body,
        grid=(_NC,),
        in_specs=[
            pl.BlockSpec((m, k), lambda c: (0, 0)),
            pl.BlockSpec((1, k, f2), lambda c: (c, 0, 0)),
        ],
        out_specs=pl.BlockSpec((1, m, f2), lambda c: (c, 0, 0)),
        out_shape=jax.ShapeDtypeStruct((_NC, m, f2), jnp.float32),
    )(x, ws)


def _mm_pe_body(x_ref, w_ref, b_ref, o_ref):
    o_ref[0] = _dot(x_ref[...], w_ref[0]) + b_ref[0]


def _mm_pe(x, w, b, block_m):
    m, k = x.shape
    f = w.shape[1]
    f2 = f // _NC
    assert m % (2 * block_m) == 0
    xp = x.reshape(m // 2, 2 * k)
    zero = jnp.zeros((k, f2), jnp.float32)
    ws = jnp.stack([
        jnp.concatenate([
            jnp.concatenate([w[:, i * f2:(i + 1) * f2], zero], axis=1),
            jnp.concatenate([zero, w[:, i * f2:(i + 1) * f2]], axis=1),
        ], axis=0)
        for i in range(_NC)
    ])
    bp = jnp.stack([jnp.tile(b[i * f2:(i + 1) * f2], 2) for i in range(_NC)])
    return pl.pallas_call(
        _mm_pe_body,
        grid=(_NC, m // 2 // block_m),
        in_specs=[
            pl.BlockSpec((block_m, 2 * k), lambda c, i: (i, 0)),
            pl.BlockSpec((1, 2 * k, 2 * f2), lambda c, i: (c, 0, 0)),
            pl.BlockSpec((1, 1, 2 * f2), lambda c, i: (c, 0, 0)),
        ],
        out_specs=pl.BlockSpec((1, block_m, 2 * f2), lambda c, i: (c, i, 0)),
        out_shape=jax.ShapeDtypeStruct((_NC, m // 2, 2 * f2), jnp.float32),
    )(xp, ws, bp.reshape(_NC, 1, 2 * f2))


def _update_body(acc_ref, deg_ref, h_ref, w2_ref, b2_ref, uw1h_ref, uw1a_ref,
                 ub1_ref, uw2_ref, ub2_ref, o_ref):
    dh = acc_ref.shape[2]
    agg = (_dot(acc_ref[0], w2_ref[0:dh, :])
           + _dot(acc_ref[1], w2_ref[dh:2 * dh, :])
           + deg_ref[...] * b2_ref[...])
    u = jnp.maximum(
        _dot(h_ref[...], uw1h_ref[...]) + _dot(agg, uw1a_ref[...])
        + ub1_ref[...], 0.0)
    h2 = _dot(u, uw2_ref[...]) + ub2_ref[...]
    o_ref[...] = jnp.maximum(h2, 0.0)


def _update(acc, deg, h, w2, b2, uw1h, uw1a, ub1, uw2, ub2, block_m):
    n, d = h.shape
    d2 = d // _NC
    f = uw2.shape[1]
    return pl.pallas_call(
        _update_body,
        grid=(n // block_m,),
        in_specs=[
            pl.BlockSpec((_NC, block_m, d2), lambda i: (0, i, 0)),
            pl.BlockSpec((block_m, 1), lambda i: (i, 0)),
            pl.BlockSpec((block_m, d), lambda i: (i, 0)),
            pl.BlockSpec((d, f), lambda i: (0, 0)),
            pl.BlockSpec((1, f), lambda i: (0, 0)),
            pl.BlockSpec((d, f), lambda i: (0, 0)),
            pl.BlockSpec((d, f), lambda i: (0, 0)),
            pl.BlockSpec((1, f), lambda i: (0, 0)),
            pl.BlockSpec((d, f), lambda i: (0, 0)),
            pl.BlockSpec((1, f), lambda i: (0, 0)),
        ],
        out_specs=pl.BlockSpec((block_m, f), lambda i: (i, 0)),
        out_shape=jax.ShapeDtypeStruct((n, f), jnp.float32),
    )(acc, deg, h, w2, b2.reshape(1, f), uw1h, uw1a, ub1.reshape(1, f),
      uw2, ub2.reshape(1, f))


def _head_body(h_ref, bidx_ref, lw1_ref, lb1_ref, lw2_ref, lb2_ref, o_ref):
    bidx = bidx_ref[...]
    gids = lax.broadcasted_iota(jnp.int32, (_G, 1), 0)
    onehot = (bidx == gids).astype(jnp.float32)
    sums = _dot(onehot, h_ref[...])
    counts = jnp.sum(onehot, axis=1, keepdims=True)
    pooled = sums / jnp.maximum(counts, 1.0)
    h2 = jnp.maximum(_dot(pooled, lw1_ref[...]) + lb1_ref[...], 0.0)
    o_ref[...] = jnp.sum(h2 * lw2_ref[...], axis=1, keepdims=True) + lb2_ref[...]


def _head(h, batch_idx, lw1, lb1, lw2, lb2):
    n, d = h.shape
    return pl.pallas_call(
        _head_body,
        out_shape=jax.ShapeDtypeStruct((_G, 1), jnp.float32),
    )(h, batch_idx.reshape(1, n), lw1, lb1.reshape(1, d),
      lw2.reshape(1, d), lb2.reshape(1, 1))


_NBUF = 2
_NIDX = 4


def _make_edge_kernel(n, e, d, chunk, compute_deg=True):
    dh = d // _NC
    e_per_t = e // _NS
    steps = e_per_t // chunk
    assert e % _NS == 0 and e_per_t % chunk == 0 and chunk % 16 == 0
    assert chunk <= 128
    assert steps % 2 == 0 and steps >= 2 * _NIDX
    rows_per_tile = (n // _NS) // 8 * 8
    tail_base = rows_per_tile * _NS
    tail = n - tail_base
    zr = 16
    assert rows_per_tile % zr == 0 and tail % 8 == 0 and tail <= zr
    groups = chunk * dh // 16
    mesh = plsc.VectorSubcoreMesh(core_axis_name="c", subcore_axis_name="s",
                                  num_cores=_NC, num_subcores=_NS)

    @functools.partial(
        pl.kernel,
        out_type=[
            jax.ShapeDtypeStruct((_NC, n, dh), jnp.float32),
            jax.ShapeDtypeStruct((n,), jnp.float32),
        ],
        mesh=mesh,
        compiler_params=pltpu.CompilerParams(use_tc_tiling_on_sc=False),
        scratch_types=(
            [pltpu.VMEM((chunk,), jnp.int32) for _ in range(_NIDX)]
            + [pltpu.VMEM((chunk,), jnp.int32) for _ in range(_NIDX)]
            + [pltpu.VMEM((chunk, dh), jnp.float32) for _ in range(2 * _NBUF)]
            + [pltpu.VMEM((chunk // 2, 2 * dh), jnp.float32) for _ in range(_NBUF)]
            + [
                pltpu.VMEM((zr, dh), jnp.float32),
                pltpu.VMEM((rows_per_tile + tail,), jnp.float32),
                pltpu.VMEM((chunk,), jnp.float32),
                pltpu.VMEM_SHARED((n, dh), jnp.float32),
                pltpu.VMEM_SHARED((n,), jnp.float32),
            ]
            + [pltpu.SemaphoreType.DMA for _ in range(3 * _NBUF + 2 * _NIDX)]
        ),
    )
    def edge_kernel(pd_hbm, ps_hbm, pe_hbm, dst_hbm, src_hbm,
                    acc_hbm, deg_hbm, *refs):
        idx_d = list(refs[0:_NIDX])
        idx_s = list(refs[_NIDX:2 * _NIDX])
        off = 2 * _NIDX
        pd_v = list(refs[off:off + _NBUF]); off += _NBUF
        ps_v = list(refs[off:off + _NBUF]); off += _NBUF
        pe_v = list(refs[off:off + _NBUF]); off += _NBUF
        zbuf, zbuf1, ones_v, acc_sh, deg_sh = refs[off:off + 5]; off += 5
        sem_pd = list(refs[off:off + _NBUF]); off += _NBUF
        sem_ps = list(refs[off:off + _NBUF]); off += _NBUF
        sem_pe = list(refs[off:off + _NBUF]); off += _NBUF
        sem_id = list(refs[off:off + _NIDX]); off += _NIDX
        sem_is = list(refs[off:off + _NIDX]); off += _NIDX

        cid = lax.axis_index("c")
        sid = lax.axis_index("s")
        pd_t = pd_hbm.at[cid]
        ps_t = ps_hbm.at[cid]
        pe_t = pe_hbm.at[cid]
        tile_base = sid * e_per_t

        zero16 = jnp.zeros((16,), jnp.float32)
        ones16 = jnp.ones((16,), jnp.float32)

        def zrow(i, _):
            def zcol(k, _):
                zbuf[i, pl.ds(k * 16, 16)] = zero16
                return 0
            return lax.fori_loop(0, dh // 16, zcol, 0)
        lax.fori_loop(0, zr, zrow, 0)

        def zones(i, _):
            ones_v[pl.ds(i * 16, 16)] = ones16
            return 0
        lax.fori_loop(0, chunk // 16, zones, 0)

        def zvec(i, _):
            zbuf1[pl.ds(i * 16, 16)] = zero16
            return 0
        lax.fori_loop(0, (rows_per_tile + tail) // 16, zvec, 0)

        base_row = sid * rows_per_tile
        for q in range(rows_per_tile // zr):
            pltpu.sync_copy(zbuf, acc_sh.at[pl.ds(base_row + q * zr, zr)])

        @pl.when((sid == _NS - 1) & (cid == 0))
        def _zero_deg_tail():
            pltpu.sync_copy(zbuf.at[pl.ds(0, tail)],
                            acc_sh.at[pl.ds(tail_base, tail)])
            pltpu.sync_copy(zbuf1,
                            deg_sh.at[pl.ds(base_row, rows_per_tile + tail)])

        @pl.when((sid == _NS - 1) & (cid == 1))
        def _zero_acc_tail():
            pltpu.sync_copy(zbuf.at[pl.ds(0, tail)],
                            acc_sh.at[pl.ds(tail_base, tail)])

        @pl.when((sid < _NS - 1) & (cid == 0))
        def _zero_deg():
            pltpu.sync_copy(zbuf1.at[pl.ds(0, rows_per_tile)],
                            deg_sh.at[pl.ds(base_row, rows_per_tile)])
        plsc.subcore_barrier()

        def idx_load(j, q):
            base = tile_base + j * chunk
            pltpu.async_copy(dst_hbm.at[pl.ds(base, chunk)], idx_d[q],
                             sem_id[q])
            pltpu.async_copy(src_hbm.at[pl.ds(base, chunk)], idx_s[q],
                             sem_is[q])

        def prep(j, b, q, load_ahead):
            base = tile_base + j * chunk
            pltpu.make_async_copy(dst_hbm.at[pl.ds(base, chunk)], idx_d[q],
                                  sem_id[q]).wait()
            pltpu.make_async_copy(src_hbm.at[pl.ds(base, chunk)], idx_s[q],
                                  sem_is[q]).wait()

            if compute_deg:
                @pl.when(cid == 0)
                def _deg():
                    pltpu.sync_copy(ones_v, deg_sh.at[idx_d[q]], add=True)
            pltpu.async_copy(pd_t.at[idx_d[q]], pd_v[b], sem_pd[b])
            pltpu.async_copy(ps_t.at[idx_s[q]], ps_v[b], sem_ps[b])
            pltpu.async_copy(pe_t.at[pl.ds(base // 2, chunk // 2)], pe_v[b],
                             sem_pe[b])
            if load_ahead:
                idx_load(j + 2, (q + 2) % _NIDX)

        def finish(j, b, q):
            base = tile_base + j * chunk
            pltpu.make_async_copy(pd_t.at[idx_d[q]], pd_v[b],
                                  sem_pd[b]).wait()
            pltpu.make_async_copy(ps_t.at[idx_s[q]], ps_v[b],
                                  sem_ps[b]).wait()
            pltpu.make_async_copy(pe_t.at[pl.ds(base // 2, chunk // 2)],
                                  pe_v[b], sem_pe[b]).wait()

            gpr = dh // 16

            @plsc.parallel_loop(0, groups, 1, unroll=8)
            def _ew(t):
                i = lax.div(t, gpr)
                k = lax.rem(t, gpr)
                s = pl.ds(k * 16, 16)
                pe = pe_v[b][lax.div(t, 2 * gpr),
                             pl.ds(lax.rem(t, 2 * gpr) * 16, 16)]
                v = pd_v[b][i, s] + ps_v[b][i, s] + pe
                pd_v[b][i, s] = jnp.maximum(v, 0.0)

            pltpu.async_copy(pd_v[b], acc_sh.at[idx_d[q]], sem_pd[b],
                             add=True)
            pltpu.make_async_copy(pd_v[b], acc_sh.at[idx_d[q]],
                                  sem_pd[b]).wait()

        assert steps % 4 == 2 and steps >= 10
        for q in range(_NIDX):
            idx_load(q, q)
        prep(0, 0, 0, False)
        prep(1, 1, 1, False)
        finish(0, 0, 0)
        prep(2, 0, 2, True)
        finish(1, 1, 1)
        prep(3, 1, 3, True)

        def main_body(jj, _):
            j0 = 2 + jj * 4
            for p in range(4):
                j = j0 + p
                b = p % 2
                q = (2 + p) % _NIDX
                finish(j, b, q)
                prep(j + 2, b, (q + 2) % _NIDX, True)
            return 0
        lax.fori_loop(0, (steps - 6) // 4, main_body, 0)

        finish(steps - 4, 0, 2)
        prep(steps - 2, 0, 0, False)
        finish(steps - 3, 1, 3)
        prep(steps - 1, 1, 1, False)
        finish(steps - 2, 0, 0)
        finish(steps - 1, 1, 1)

        plsc.subcore_barrier()
        pltpu.sync_copy(acc_sh.at[pl.ds(base_row, rows_per_tile)],
                        acc_hbm.at[cid, pl.ds(base_row, rows_per_tile)])

        @pl.when(sid == _NS - 1)
        def _write_tail():
            pltpu.sync_copy(acc_sh.at[pl.ds(tail_base, tail)],
                            acc_hbm.at[cid, pl.ds(tail_base, tail)])

        @pl.when((sid == _NS - 1) & (cid == 0))
        def _write_deg_tail():
            pltpu.sync_copy(deg_sh.at[pl.ds(base_row, rows_per_tile + tail)],
                            zbuf1)
            pltpu.sync_copy(zbuf1,
                            deg_hbm.at[pl.ds(base_row, rows_per_tile + tail)])

        @pl.when((sid < _NS - 1) & (cid == 0))
        def _write_deg():
            pltpu.sync_copy(deg_sh.at[pl.ds(base_row, rows_per_tile)],
                            zbuf1.at[pl.ds(0, rows_per_tile)])
            pltpu.sync_copy(zbuf1.at[pl.ds(0, rows_per_tile)],
                            deg_hbm.at[pl.ds(base_row, rows_per_tile)])

    return edge_kernel


def kernel(x, edge_index, edge_attr, batch_idx,
           m0_w1, m0_b1, m0_w2, m0_b2, u0_w1, u0_b1, u0_w2, u0_b2,
           m1_w1, m1_b1, m1_w2, m1_b2, u1_w1, u1_b1, u1_w2, u1_b2,
           lin1_w, lin1_b, lin2_w, lin2_b):
    n, d = x.shape
    e = edge_index.shape[1]
    h_dim = m0_w2.shape[0]
    src = edge_index[0]
    dst = edge_index[1]

    edge_fn0 = _make_edge_kernel(n, e, h_dim, chunk=80)
    edge_fn1 = _make_edge_kernel(n, e, h_dim, chunk=80, compute_deg=False)

    def conv(h, mw1, mb1, mw2, mb2, uw1, ub1, uw2, ub2, deg1, edge_fn):
        hd = h.shape[1]
        pd = _mm_split(h, mw1[:hd])
        ps = _mm_split(h, mw1[hd:2 * hd])
        pe = _mm_pe(edge_attr, mw1[2 * hd:], mb1, block_m=8000)
        acc, deg = edge_fn(pd, ps, pe, dst, src)
        if deg1 is None:
            deg1 = deg.reshape(n, 1)
        h_new = _update(acc, deg1, h, mw2, mb2, uw1[:hd], uw1[hd:],
                        ub1, uw2, ub2, block_m=2000)
        return h_new, deg1

    h1, deg1 = conv(x, m0_w1, m0_b1, m0_w2, m0_b2,
                    u0_w1, u0_b1, u0_w2, u0_b2, None, edge_fn0)
    h2, _ = conv(h1, m1_w1, m1_b1, m1_w2, m1_b2,
                 u1_w1, u1_b1, u1_w2, u1_b2, deg1, edge_fn1)
    return _head(h2, batch_idx, lin1_w, lin1_b, lin2_w, lin2_b)
